# Initial kernel scaffold; baseline (speedup 1.0000x reference)
import jax
import jax.numpy as jnp
from jax.experimental import pallas as pl


def kernel(node_feat, feat, edge_index, W_conv, b_conv, W_lin, b_lin):
    raise NotImplementedError("write your pallas kernel here")


# speedup vs baseline: 11.4222x; 11.4222x over previous
import jax
import jax.numpy as jnp
from jax import lax
from jax.experimental import pallas as pl
from jax.experimental.pallas import tpu as pltpu
from jax.experimental.pallas import tpu_sc as plsc

NC, NS = 2, 16
NW = NC * NS
LANES = 128
NB = 8
D = 18
DP = 24


def _fill_f32(ref, nrow16, value):
    for k in range(nrow16):
        ref[pl.ds(k * 16, 16)] = jnp.full((16,), value, jnp.float32)


def _deg_body(src2d, dst2d, od_out, id_out, *rest):
    sidx = rest[:NB]
    didx = rest[NB:2 * NB]
    ones_v, zv, odeg, ideg, sem_i, sem_a = rest[2 * NB:]
    c = lax.axis_index("c")
    s = lax.axis_index("s")
    w = s * NC + c
    nrows = src2d.shape[0]
    nsc = nrows // NB
    pad = odeg.shape[0] // NS
    _fill_f32(ones_v, LANES // 16, 1.0)
    _fill_f32(zv, LANES // 16, 0.0)

    def zbody(k, carry):
        pltpu.sync_copy(zv, odeg.at[pl.ds(s * pad + k * LANES, LANES)])
        pltpu.sync_copy(zv, ideg.at[pl.ds(s * pad + k * LANES, LANES)])
        return carry

    lax.fori_loop(0, pad // LANES, zbody, 0)
    rem = pad % LANES
    if rem:
        pltpu.sync_copy(
            zv.at[pl.ds(0, rem)], odeg.at[pl.ds(s * pad + pad - rem, rem)]
        )
        pltpu.sync_copy(
            zv.at[pl.ds(0, rem)], ideg.at[pl.ds(s * pad + pad - rem, rem)]
        )
    plsc.subcore_barrier()
    iters = (nsc + NW - 1) // NW

    def body(i, carry):
        scid = w + NW * i

        @pl.when(scid < nsc)
        def _():
            base = scid * NB
            lds = [
                pltpu.async_copy(src2d.at[base + b], sidx[b], sem_i)
                for b in range(NB)
            ] + [
                pltpu.async_copy(dst2d.at[base + b], didx[b], sem_i)
                for b in range(NB)
            ]
            for d_ in lds:
                d_.wait()
            adds = [
                pltpu.async_copy(ones_v, odeg.at[sidx[b]], sem_a, add=True)
                for b in range(NB)
            ] + [
                pltpu.async_copy(ones_v, ideg.at[didx[b]], sem_a, add=True)
                for b in range(NB)
            ]
            for d_ in adds:
                d_.wait()

        return carry

    lax.fori_loop(0, iters, body, 0)
    plsc.subcore_barrier()
    pltpu.sync_copy(odeg.at[pl.ds(s * pad, pad)], od_out.at[c, pl.ds(s * pad, pad)])
    pltpu.sync_copy(ideg.at[pl.ds(s * pad, pad)], id_out.at[c, pl.ds(s * pad, pad)])


def _agg_body(src2d, dstab, hw, zsrc, agg_out, *rest):
    sidx = rest[:NB]
    didx = rest[NB:2 * NB]
    rows = rest[2 * NB:3 * NB]
    zrow, agg, sem_i, sem_g, sem_a = rest[3 * NB:]
    c = lax.axis_index("c")
    s = lax.axis_index("s")
    nrows = src2d.shape[0]
    nsc = nrows // NB
    npt = agg.shape[0] // NS
    pltpu.sync_copy(zsrc, zrow)

    def zbody(k, carry):
        pltpu.sync_copy(zrow, agg.at[pl.ds(s * npt + k * LANES, LANES)])
        return carry

    lax.fori_loop(0, npt // LANES, zbody, 0)
    plsc.subcore_barrier()
    iters = (nsc + NS - 1) // NS

    def body(i, carry):
        scid = s + NS * i

        @pl.when(scid < nsc)
        def _():
            base = scid * NB
            lds = [
                pltpu.async_copy(src2d.at[base + b], sidx[b], sem_i)
                for b in range(NB)
            ] + [
                pltpu.async_copy(dstab.at[c * nrows + base + b], didx[b], sem_i)
                for b in range(NB)
            ]
            for d_ in lds:
                d_.wait()
            gts = [
                pltpu.async_copy(hw.at[sidx[b]], rows[b], sem_g)
                for b in range(NB)
            ]
            for d_ in gts:
                d_.wait()
            adds = [
                pltpu.async_copy(rows[b], agg.at[didx[b]], sem_a, add=True)
                for b in range(NB)
            ]
            for d_ in adds:
                d_.wait()

        return carry

    lax.fori_loop(0, iters, body, 0)
    plsc.subcore_barrier()
    pltpu.sync_copy(agg.at[pl.ds(s * npt, npt)], agg_out.at[c, pl.ds(s * npt, npt)])


def _dense1_body(nf_ref, odp_ref, feat_ref, wc_ref, wl_ref, bl_ref, hw_ref, tra_ref):
    od = odp_ref[0, :] + odp_ref[1, :]
    nrm = lax.rsqrt(jnp.maximum(od, 1.0))
    hw = jnp.dot(nf_ref[:, :], wc_ref[:, :], preferred_element_type=jnp.float32)
    hw_ref[:, :] = hw * nrm[:, None]
    tra_ref[:, :] = (
        jnp.dot(feat_ref[:, :], wl_ref[:, :], preferred_element_type=jnp.float32)
        + bl_ref[:, :]
    )


def _remap_body(hn_ref, dst_ref, out_ref):
    hn = hn_ref[0]
    trash = hn_ref[1]
    d_ = dst_ref[:, :]
    out_ref[0] = jnp.where(d_ < hn, d_, trash)
    out_ref[1] = jnp.where(d_ >= hn, d_ - hn, trash)


def _dense2_body(agg_ref, idp_ref, tra_ref, bc_ref, out_ref):
    indeg = idp_ref[0, :] + idp_ref[1, :]
    nrm = lax.rsqrt(jnp.maximum(indeg, 1.0))
    rst = jnp.maximum(agg_ref[:, :] * nrm[:, None] + bc_ref[:, :], 0.0)
    out_ref[:, :] = jnp.concatenate([tra_ref[:, :], rst], axis=1)


def kernel(node_feat, feat, edge_index, W_conv, b_conv, W_lin, b_lin):
    N = node_feat.shape[0]
    E = edge_index.shape[1]
    H = W_lin.shape[1]
    src2d = edge_index[0].astype(jnp.int32).reshape(E // LANES, LANES)
    dst2d = edge_index[1].astype(jnp.int32).reshape(E // LANES, LANES)
    nrows = E // LANES

    pad_tile = ((N + NS - 1) // NS + 7) // 8 * 8
    padn = NS * pad_tile
    HN = ((N + 1) // 2 + 7) // 8 * 8
    ACC = ((HN + 1) + NS * LANES - 1) // (NS * LANES) * (NS * LANES)

    mesh = plsc.VectorSubcoreMesh(
        core_axis_name="c", subcore_axis_name="s", num_cores=NC, num_subcores=NS
    )
    sc_params = pltpu.CompilerParams(use_tc_tiling_on_sc=False)

    deg_call = pl.kernel(
        _deg_body,
        out_type=[
            jax.ShapeDtypeStruct((NC, padn), jnp.float32),
            jax.ShapeDtypeStruct((NC, padn), jnp.float32),
        ],
        mesh=mesh,
        compiler_params=sc_params,
        scratch_types=[pltpu.VMEM((LANES,), jnp.int32) for _ in range(2 * NB)]
        + [
            pltpu.VMEM((LANES,), jnp.float32),
            pltpu.VMEM((LANES,), jnp.float32),
            pltpu.VMEM_SHARED((padn,), jnp.float32),
            pltpu.VMEM_SHARED((padn,), jnp.float32),
            pltpu.SemaphoreType.DMA,
            pltpu.SemaphoreType.DMA,
        ],
    )
    odp, idp = deg_call(src2d, dst2d)
    odp = odp[:, :N]
    idp = idp[:, :N]

    wc_pad = jnp.concatenate(
        [W_conv, jnp.zeros((D, DP - D), jnp.float32)], axis=1
    )
    R = 2048
    nb_ = (N + R - 1) // R
    hw, tra = pl.pallas_call(
        _dense1_body,
        grid=(nb_,),
        in_specs=[
            pl.BlockSpec((R, D), lambda i: (i, 0)),
            pl.BlockSpec((NC, R), lambda i: (0, i)),
            pl.BlockSpec((R, feat.shape[1]), lambda i: (i, 0)),
            pl.BlockSpec((D, DP), lambda i: (0, 0)),
            pl.BlockSpec((feat.shape[1], H), lambda i: (0, 0)),
            pl.BlockSpec((1, H), lambda i: (0, 0)),
        ],
        out_specs=[
            pl.BlockSpec((R, DP), lambda i: (i, 0)),
            pl.BlockSpec((R, H), lambda i: (i, 0)),
        ],
        out_shape=[
            jax.ShapeDtypeStruct((N, DP), jnp.float32),
            jax.ShapeDtypeStruct((N, H), jnp.float32),
        ],
    )(node_feat, odp, feat, wc_pad, W_lin, b_lin.reshape(1, H))

    RB = 1000
    nrb = (nrows + RB - 1) // RB
    dstab = pl.pallas_call(
        _remap_body,
        grid=(nrb,),
        in_specs=[
            pl.BlockSpec(memory_space=pltpu.SMEM),
            pl.BlockSpec((RB, LANES), lambda i: (i, 0)),
        ],
        out_specs=pl.BlockSpec((NC, RB, LANES), lambda i: (0, i, 0)),
        out_shape=jax.ShapeDtypeStruct((NC, nrows, LANES), jnp.int32),
    )(jnp.array([HN, HN], jnp.int32), dst2d)
    dstab = dstab.reshape(NC * nrows, LANES)

    agg_call = pl.kernel(
        _agg_body,
        out_type=jax.ShapeDtypeStruct((NC, ACC, DP), jnp.float32),
        mesh=mesh,
        compiler_params=sc_params,
        scratch_types=[pltpu.VMEM((LANES,), jnp.int32) for _ in range(2 * NB)]
        + [pltpu.VMEM((LANES, DP), jnp.float32) for _ in range(NB)]
        + [
            pltpu.VMEM((LANES, DP), jnp.float32),
            pltpu.VMEM_SHARED((ACC, DP), jnp.float32),
            pltpu.SemaphoreType.DMA,
            pltpu.SemaphoreType.DMA,
            pltpu.SemaphoreType.DMA,
        ],
    )
    aggp = agg_call(src2d, dstab, hw, jnp.zeros((LANES, DP), jnp.float32))
    agg = jnp.concatenate([aggp[0, :HN], aggp[1, : N - HN]], axis=0)

    bc_pad = jnp.concatenate(
        [b_conv, jnp.zeros((DP - D,), jnp.float32)]
    ).reshape(1, DP)
    out = pl.pallas_call(
        _dense2_body,
        grid=(nb_,),
        in_specs=[
            pl.BlockSpec((R, DP), lambda i: (i, 0)),
            pl.BlockSpec((NC, R), lambda i: (0, i)),
            pl.BlockSpec((R, H), lambda i: (i, 0)),
            pl.BlockSpec((1, DP), lambda i: (0, 0)),
        ],
        out_specs=pl.BlockSpec((R, H + D), lambda i: (i, 0)),
        out_shape=jax.ShapeDtypeStruct((N, H + D), jnp.float32),
    )(agg, idp, tra, bc_pad)
    return out
